# baseline (speedup 1.0000x reference)
import functools

import jax
import jax.numpy as jnp
from jax import lax
from jax.experimental import pallas as pl
from jax.experimental.pallas import tpu as pltpu
from jax.experimental.pallas import tpu_sc as plsc

_N = 10000
_E = 320000
_D = 128
_B = 128
_C = 10

_NC = 2
_NS = 16
_NW = _NC * _NS

_APAD = 10240
_ROWS_PER_TILE = _APAD // _NS
_ZC = 80
_EDGES_PER_TILE = _E // _NW
_CHUNK = 80
_NCHUNK = _EDGES_PER_TILE // _CHUNK

_RB = 1000
_NBLK = _N // _RB


_NBUF = 4
_LEAD = 3


def _sc_body(x_hbm, src_hbm, dst_hbm, out_hbm, agg_sh, sidx, didx, rows,
             gsem, ssem, sisem, disem):
    c = lax.axis_index("c")
    s = lax.axis_index("s")
    wid = c * _NS + s

    zeros16 = jnp.zeros((16,), jnp.float32)

    def zrow(r, carry):
        for k in range(_D // 16):
            rows[_NBUF - 1, r, pl.ds(k * 16, 16)] = zeros16
        return carry

    lax.fori_loop(0, _ZC, zrow, 0)
    zbuf = rows.at[_NBUF - 1]
    for k in range(_ROWS_PER_TILE // _ZC):
        pltpu.async_copy(zbuf, agg_sh.at[pl.ds(s * _ROWS_PER_TILE + k * _ZC, _ZC)],
                         ssem.at[k % _NBUF])
    for b in range(_NBUF):
        pltpu.async_copy(src_hbm.at[wid, b], sidx.at[b], sisem.at[b])
    for b in range(_LEAD):
        pltpu.async_copy(dst_hbm.at[wid, b], didx.at[b], disem.at[b])
    for b in range(_LEAD):
        pltpu.make_async_copy(src_hbm.at[wid, b], sidx.at[b], sisem.at[b]).wait()
        pltpu.async_copy(x_hbm.at[sidx.at[b]], rows.at[b], gsem.at[b])
    for k in range(_ROWS_PER_TILE // _ZC):
        pltpu.make_async_copy(zbuf, agg_sh.at[pl.ds(0, _ZC)], ssem.at[k % _NBUF]).wait()
    plsc.subcore_barrier()

    idescr = src_hbm.at[0, 0]
    def outer(g, carry):
        for b in range(_NBUF):
            j = g * _NBUF + b
            b2 = (b + _LEAD) % _NBUF
            pltpu.make_async_copy(x_hbm.at[sidx.at[b]], rows.at[b], gsem.at[b]).wait()

            @pl.when(j + _NBUF < _NCHUNK)
            def _():
                pltpu.async_copy(src_hbm.at[wid, j + _NBUF], sidx.at[b], sisem.at[b])
            pltpu.make_async_copy(idescr, didx.at[b], disem.at[b]).wait()
            pltpu.async_copy(rows.at[b], agg_sh.at[didx.at[b]], ssem.at[b], add=True)

            @pl.when(j >= _NBUF - _LEAD)
            def _():
                pltpu.make_async_copy(rows.at[b2], agg_sh.at[didx.at[0]],
                                      ssem.at[b2]).wait()

            @pl.when(j + _LEAD < _NCHUNK)
            def _():
                pltpu.async_copy(dst_hbm.at[wid, j + _LEAD], didx.at[b2], disem.at[b2])
                pltpu.make_async_copy(idescr, sidx.at[b2], sisem.at[b2]).wait()
                pltpu.async_copy(x_hbm.at[sidx.at[b2]], rows.at[b2], gsem.at[b2])
        return carry

    lax.fori_loop(0, (_NCHUNK - 1) // _NBUF, outer, 0)
    pltpu.make_async_copy(x_hbm.at[sidx.at[0]], rows.at[0], gsem.at[0]).wait()
    pltpu.make_async_copy(idescr, didx.at[0], disem.at[0]).wait()
    pltpu.async_copy(rows.at[0], agg_sh.at[didx.at[0]], ssem.at[0], add=True)
    pltpu.make_async_copy(rows.at[3], agg_sh.at[didx.at[0]], ssem.at[3]).wait()
    pltpu.make_async_copy(rows.at[0], agg_sh.at[didx.at[0]], ssem.at[0]).wait()
    plsc.subcore_barrier()

    pltpu.sync_copy(
        agg_sh.at[pl.ds(s * _ROWS_PER_TILE, _ROWS_PER_TILE)],
        out_hbm.at[c, pl.ds(s * _ROWS_PER_TILE, _ROWS_PER_TILE)],
    )


_sc_gather_scatter = functools.partial(
    pl.kernel,
    out_type=jax.ShapeDtypeStruct((_NC, _APAD, _D), jnp.float32),
    mesh=plsc.VectorSubcoreMesh(core_axis_name="c", subcore_axis_name="s"),
    scratch_types=[
        pltpu.VMEM_SHARED((_APAD, _D), jnp.float32),
        pltpu.VMEM((_NBUF, _CHUNK), jnp.int32),
        pltpu.VMEM((_NBUF, _CHUNK), jnp.int32),
        pltpu.VMEM((_NBUF, _CHUNK, _D), jnp.float32),
        pltpu.SemaphoreType.DMA((_NBUF,)),
        pltpu.SemaphoreType.DMA((_NBUF,)),
        pltpu.SemaphoreType.DMA((_NBUF,)),
        pltpu.SemaphoreType.DMA((_NBUF,)),
    ],
)(_sc_body)


def _mlp_body(x_ref, a_ref, w1_ref, b1_ref, w2_ref, b2_ref, o_ref, *, out_relu):
    a = x_ref[...] + a_ref[0] + a_ref[1]
    h = jnp.maximum(
        jnp.dot(a, w1_ref[...], preferred_element_type=jnp.float32) + b1_ref[...], 0.0)
    o = jnp.dot(h, w2_ref[...], preferred_element_type=jnp.float32) + b2_ref[...]
    if out_relu:
        o = jnp.maximum(o, 0.0)
    o_ref[...] = o


def _mlp(x, agg, w1, b1, w2, b2, out_relu):
    big = pl.BlockSpec((_RB, _D), lambda i: (i, 0))
    aspec = pl.BlockSpec((_NC, _RB, _D), lambda i: (0, i, 0))
    wspec = pl.BlockSpec((_D, _D), lambda i: (0, 0))
    bspec = pl.BlockSpec((1, _D), lambda i: (0, 0))
    return pl.pallas_call(
        functools.partial(_mlp_body, out_relu=out_relu),
        grid=(_NBLK,),
        in_specs=[big, aspec, wspec, bspec, wspec, bspec],
        out_specs=big,
        out_shape=jax.ShapeDtypeStruct((_N, _D), jnp.float32),
    )(x, agg, w1, b1.reshape(1, _D), w2, b2.reshape(1, _D))


def _final_body(x_ref, a_ref, w1_ref, b1_ref, w2_ref, b2_ref, bidx_ref,
                h1w_ref, h1b_ref, h2w_ref, h2b_ref, o_ref, acc, cnt):
    j = pl.program_id(0)
    a = x_ref[...] + a_ref[0] + a_ref[1]
    h = jnp.maximum(
        jnp.dot(a, w1_ref[...], preferred_element_type=jnp.float32) + b1_ref[...], 0.0)
    h3 = jnp.dot(h, w2_ref[...], preferred_element_type=jnp.float32) + b2_ref[...]

    bi = bidx_ref[0, 0, :]
    gid = lax.broadcasted_iota(jnp.int32, (_RB, _B), 1)
    oh = (bi[:, None] == gid).astype(jnp.float32)

    @pl.when(j == 0)
    def _():
        acc[...] = jnp.zeros_like(acc)
        cnt[...] = jnp.zeros_like(cnt)

    acc[...] += lax.dot_general(oh, h3, (((0,), (0,)), ((), ())),
                                preferred_element_type=jnp.float32)
    cnt[...] += jnp.sum(oh, axis=0, keepdims=True)

    @pl.when(j == pl.num_programs(0) - 1)
    def _():
        recip = 1.0 / jnp.clip(cnt[...], 1.0, None)
        eye = (lax.broadcasted_iota(jnp.int32, (_B, _B), 0) ==
               lax.broadcasted_iota(jnp.int32, (_B, _B), 1)).astype(jnp.float32)
        diag = eye * recip
        pooled = jnp.dot(diag, acc[...], preferred_element_type=jnp.float32)
        z = jnp.maximum(
            jnp.dot(pooled, h1w_ref[...], preferred_element_type=jnp.float32)
            + h1b_ref[...], 0.0)
        o_ref[...] = (jnp.dot(z, h2w_ref[...], preferred_element_type=jnp.float32)
                      + h2b_ref[...])


def _final(x, agg, w1, b1, w2, b2, bidx, h1w, h1b, h2w_pad, h2b_pad):
    big = pl.BlockSpec((_RB, _D), lambda i: (i, 0))
    aspec = pl.BlockSpec((_NC, _RB, _D), lambda i: (0, i, 0))
    wspec = pl.BlockSpec((_D, _D), lambda i: (0, 0))
    bspec = pl.BlockSpec((1, _D), lambda i: (0, 0))
    bidx_spec = pl.BlockSpec((1, 1, _RB), lambda i: (i, 0, 0))
    ospec = pl.BlockSpec((_B, _B), lambda i: (0, 0))
    return pl.pallas_call(
        _final_body,
        grid=(_NBLK,),
        in_specs=[big, aspec, wspec, bspec, wspec, bspec, bidx_spec,
                  wspec, bspec, wspec, bspec],
        out_specs=ospec,
        out_shape=jax.ShapeDtypeStruct((_B, _B), jnp.float32),
        scratch_shapes=[pltpu.VMEM((_B, _B), jnp.float32),
                        pltpu.VMEM((1, _B), jnp.float32)],
    )(x, agg, w1, b1.reshape(1, _D), w2, b2.reshape(1, _D), bidx,
      h1w, h1b.reshape(1, _D), h2w_pad, h2b_pad.reshape(1, _B))


def kernel(x, edge_index, batch_sample_indices,
           l1W1, l1b1, l1W2, l1b2,
           l2W1, l2b1, l2W2, l2b2,
           l3W1, l3b1, l3W2, l3b2,
           h1W, h1b, h2W, h2b):
    src = edge_index[0].reshape(_NW, _NCHUNK, _CHUNK)
    dst = edge_index[1].reshape(_NW, _NCHUNK, _CHUNK)
    bidx = batch_sample_indices.reshape(_NBLK, 1, _RB)
    h2w_pad = jnp.pad(h2W, ((0, 0), (0, _B - _C)))
    h2b_pad = jnp.pad(h2b, (0, _B - _C))

    agg = _sc_gather_scatter(x, src, dst)
    h = _mlp(x, agg, l1W1, l1b1, l1W2, l1b2, True)
    agg = _sc_gather_scatter(h, src, dst)
    h = _mlp(h, agg, l2W1, l2b1, l2W2, l2b2, True)
    agg = _sc_gather_scatter(h, src, dst)
    out = _final(h, agg, l3W1, l3b1, l3W2, l3b2, bidx, h1W, h1b, h2w_pad, h2b_pad)
    return out[:, :_C]


# speedup vs baseline: 1.0081x; 1.0081x over previous
import functools

import jax
import jax.numpy as jnp
from jax import lax
from jax.experimental import pallas as pl
from jax.experimental.pallas import tpu as pltpu
from jax.experimental.pallas import tpu_sc as plsc

_N = 10000
_E = 320000
_D = 128
_B = 128
_C = 10

_NC = 2
_NS = 16
_NW = _NC * _NS

_APAD = 10240
_ROWS_PER_TILE = _APAD // _NS
_ZC = 80
_EDGES_PER_TILE = _E // _NW
_CHUNK = 50
_NCHUNK = _EDGES_PER_TILE // _CHUNK

_RB = 1000
_NBLK = _N // _RB


_NBUF = 5
_LEAD = 4


def _sc_body(x_hbm, src_hbm, dst_hbm, out_hbm, agg_sh, zbuf, sidx, didx, rows,
             gsem, ssem, sisem, disem):
    c = lax.axis_index("c")
    s = lax.axis_index("s")
    wid = c * _NS + s

    zeros16 = jnp.zeros((16,), jnp.float32)

    def zrow(r, carry):
        for k in range(_D // 16):
            zbuf[r, pl.ds(k * 16, 16)] = zeros16
        return carry

    lax.fori_loop(0, _ZC, zrow, 0)
    for k in range(_ROWS_PER_TILE // _ZC):
        pltpu.async_copy(zbuf, agg_sh.at[pl.ds(s * _ROWS_PER_TILE + k * _ZC, _ZC)],
                         ssem.at[k % _NBUF])
    for b in range(_NBUF):
        pltpu.async_copy(src_hbm.at[wid, b], sidx.at[b], sisem.at[b])
    for b in range(_LEAD):
        pltpu.async_copy(dst_hbm.at[wid, b], didx.at[b], disem.at[b])
    for b in range(_LEAD):
        pltpu.make_async_copy(src_hbm.at[wid, b], sidx.at[b], sisem.at[b]).wait()
        pltpu.async_copy(x_hbm.at[sidx.at[b]], rows.at[b], gsem.at[b])
    for k in range(_ROWS_PER_TILE // _ZC):
        pltpu.make_async_copy(zbuf, agg_sh.at[pl.ds(0, _ZC)], ssem.at[k % _NBUF]).wait()
    plsc.subcore_barrier()

    idescr = src_hbm.at[0, 0]
    def outer(g, carry):
        for b in range(_NBUF):
            j = g * _NBUF + b
            b2 = (b + _LEAD) % _NBUF
            pltpu.make_async_copy(x_hbm.at[sidx.at[b]], rows.at[b], gsem.at[b]).wait()

            @pl.when(j + _NBUF < _NCHUNK)
            def _():
                pltpu.async_copy(src_hbm.at[wid, j + _NBUF], sidx.at[b], sisem.at[b])
            pltpu.make_async_copy(idescr, didx.at[b], disem.at[b]).wait()
            pltpu.async_copy(rows.at[b], agg_sh.at[didx.at[b]], ssem.at[b], add=True)

            @pl.when(j >= _NBUF - _LEAD)
            def _():
                pltpu.make_async_copy(rows.at[b2], agg_sh.at[didx.at[0]],
                                      ssem.at[b2]).wait()

            @pl.when(j + _LEAD < _NCHUNK)
            def _():
                pltpu.async_copy(dst_hbm.at[wid, j + _LEAD], didx.at[b2], disem.at[b2])
                pltpu.make_async_copy(idescr, sidx.at[b2], sisem.at[b2]).wait()
                pltpu.async_copy(x_hbm.at[sidx.at[b2]], rows.at[b2], gsem.at[b2])
        return carry

    lax.fori_loop(0, _NCHUNK // _NBUF, outer, 0)
    for k in range(_NBUF - _LEAD):
        b = (_NCHUNK - (_NBUF - _LEAD) + k) % _NBUF
        pltpu.make_async_copy(rows.at[b], agg_sh.at[didx.at[0]], ssem.at[b]).wait()
    plsc.subcore_barrier()

    pltpu.sync_copy(
        agg_sh.at[pl.ds(s * _ROWS_PER_TILE, _ROWS_PER_TILE)],
        out_hbm.at[c, pl.ds(s * _ROWS_PER_TILE, _ROWS_PER_TILE)],
    )


_sc_gather_scatter = functools.partial(
    pl.kernel,
    out_type=jax.ShapeDtypeStruct((_NC, _APAD, _D), jnp.float32),
    mesh=plsc.VectorSubcoreMesh(core_axis_name="c", subcore_axis_name="s"),
    scratch_types=[
        pltpu.VMEM_SHARED((_APAD, _D), jnp.float32),
        pltpu.VMEM((_ZC, _D), jnp.float32),
        pltpu.VMEM((_NBUF, _CHUNK), jnp.int32),
        pltpu.VMEM((_NBUF, _CHUNK), jnp.int32),
        pltpu.VMEM((_NBUF, _CHUNK, _D), jnp.float32),
        pltpu.SemaphoreType.DMA((_NBUF,)),
        pltpu.SemaphoreType.DMA((_NBUF,)),
        pltpu.SemaphoreType.DMA((_NBUF,)),
        pltpu.SemaphoreType.DMA((_NBUF,)),
    ],
)(_sc_body)


def _mlp_body(x_ref, a_ref, w1_ref, b1_ref, w2_ref, b2_ref, o_ref, *, out_relu):
    a = x_ref[...] + a_ref[0] + a_ref[1]
    h = jnp.maximum(
        jnp.dot(a, w1_ref[...], preferred_element_type=jnp.float32) + b1_ref[...], 0.0)
    o = jnp.dot(h, w2_ref[...], preferred_element_type=jnp.float32) + b2_ref[...]
    if out_relu:
        o = jnp.maximum(o, 0.0)
    o_ref[...] = o


def _mlp(x, agg, w1, b1, w2, b2, out_relu):
    big = pl.BlockSpec((_RB, _D), lambda i: (i, 0))
    aspec = pl.BlockSpec((_NC, _RB, _D), lambda i: (0, i, 0))
    wspec = pl.BlockSpec((_D, _D), lambda i: (0, 0))
    bspec = pl.BlockSpec((1, _D), lambda i: (0, 0))
    return pl.pallas_call(
        functools.partial(_mlp_body, out_relu=out_relu),
        grid=(_NBLK,),
        in_specs=[big, aspec, wspec, bspec, wspec, bspec],
        out_specs=big,
        out_shape=jax.ShapeDtypeStruct((_N, _D), jnp.float32),
    )(x, agg, w1, b1.reshape(1, _D), w2, b2.reshape(1, _D))


def _final_body(x_ref, a_ref, w1_ref, b1_ref, w2_ref, b2_ref, bidx_ref,
                h1w_ref, h1b_ref, h2w_ref, h2b_ref, o_ref, acc, cnt):
    j = pl.program_id(0)
    a = x_ref[...] + a_ref[0] + a_ref[1]
    h = jnp.maximum(
        jnp.dot(a, w1_ref[...], preferred_element_type=jnp.float32) + b1_ref[...], 0.0)
    h3 = jnp.dot(h, w2_ref[...], preferred_element_type=jnp.float32) + b2_ref[...]

    bi = bidx_ref[0, 0, :]
    gid = lax.broadcasted_iota(jnp.int32, (_RB, _B), 1)
    oh = (bi[:, None] == gid).astype(jnp.float32)

    @pl.when(j == 0)
    def _():
        acc[...] = jnp.zeros_like(acc)
        cnt[...] = jnp.zeros_like(cnt)

    acc[...] += lax.dot_general(oh, h3, (((0,), (0,)), ((), ())),
                                preferred_element_type=jnp.float32)
    cnt[...] += jnp.sum(oh, axis=0, keepdims=True)

    @pl.when(j == pl.num_programs(0) - 1)
    def _():
        recip = 1.0 / jnp.clip(cnt[...], 1.0, None)
        eye = (lax.broadcasted_iota(jnp.int32, (_B, _B), 0) ==
               lax.broadcasted_iota(jnp.int32, (_B, _B), 1)).astype(jnp.float32)
        diag = eye * recip
        pooled = jnp.dot(diag, acc[...], preferred_element_type=jnp.float32)
        z = jnp.maximum(
            jnp.dot(pooled, h1w_ref[...], preferred_element_type=jnp.float32)
            + h1b_ref[...], 0.0)
        o_ref[...] = (jnp.dot(z, h2w_ref[...], preferred_element_type=jnp.float32)
                      + h2b_ref[...])


def _final(x, agg, w1, b1, w2, b2, bidx, h1w, h1b, h2w_pad, h2b_pad):
    big = pl.BlockSpec((_RB, _D), lambda i: (i, 0))
    aspec = pl.BlockSpec((_NC, _RB, _D), lambda i: (0, i, 0))
    wspec = pl.BlockSpec((_D, _D), lambda i: (0, 0))
    bspec = pl.BlockSpec((1, _D), lambda i: (0, 0))
    bidx_spec = pl.BlockSpec((1, 1, _RB), lambda i: (i, 0, 0))
    ospec = pl.BlockSpec((_B, _B), lambda i: (0, 0))
    return pl.pallas_call(
        _final_body,
        grid=(_NBLK,),
        in_specs=[big, aspec, wspec, bspec, wspec, bspec, bidx_spec,
                  wspec, bspec, wspec, bspec],
        out_specs=ospec,
        out_shape=jax.ShapeDtypeStruct((_B, _B), jnp.float32),
        scratch_shapes=[pltpu.VMEM((_B, _B), jnp.float32),
                        pltpu.VMEM((1, _B), jnp.float32)],
    )(x, agg, w1, b1.reshape(1, _D), w2, b2.reshape(1, _D), bidx,
      h1w, h1b.reshape(1, _D), h2w_pad, h2b_pad.reshape(1, _B))


def kernel(x, edge_index, batch_sample_indices,
           l1W1, l1b1, l1W2, l1b2,
           l2W1, l2b1, l2W2, l2b2,
           l3W1, l3b1, l3W2, l3b2,
           h1W, h1b, h2W, h2b):
    src = edge_index[0].reshape(_NW, _NCHUNK, _CHUNK)
    dst = edge_index[1].reshape(_NW, _NCHUNK, _CHUNK)
    bidx = batch_sample_indices.reshape(_NBLK, 1, _RB)
    h2w_pad = jnp.pad(h2W, ((0, 0), (0, _B - _C)))
    h2b_pad = jnp.pad(h2b, (0, _B - _C))

    agg = _sc_gather_scatter(x, src, dst)
    h = _mlp(x, agg, l1W1, l1b1, l1W2, l1b2, True)
    agg = _sc_gather_scatter(h, src, dst)
    h = _mlp(h, agg, l2W1, l2b1, l2W2, l2b2, True)
    agg = _sc_gather_scatter(h, src, dst)
    out = _final(h, agg, l3W1, l3b1, l3W2, l3b2, bidx, h1W, h1b, h2w_pad, h2b_pad)
    return out[:, :_C]


# speedup vs baseline: 1.0371x; 1.0288x over previous
import functools

import jax
import jax.numpy as jnp
from jax import lax
from jax.experimental import pallas as pl
from jax.experimental.pallas import tpu as pltpu
from jax.experimental.pallas import tpu_sc as plsc

_N = 10000
_E = 320000
_D = 128
_B = 128
_C = 10

_NC = 2
_NS = 16
_NW = _NC * _NS

_APAD = 10240
_ROWS_PER_TILE = _APAD // _NS
_ZC = 80
_EDGES_PER_TILE = _E // _NW
_CHUNK = 50
_NCHUNK = _EDGES_PER_TILE // _CHUNK

_RB = 2000
_NBLK = _N // _RB


_NBUF = 5
_LEAD = 4


def _sc_body(x_hbm, src_hbm, dst_hbm, out_hbm, agg_sh, zbuf, sidx, didx, rows,
             gsem, ssem, sisem, disem):
    c = lax.axis_index("c")
    s = lax.axis_index("s")
    wid = c * _NS + s

    zeros16 = jnp.zeros((16,), jnp.float32)

    def zrow(r, carry):
        for k in range(_D // 16):
            zbuf[r, pl.ds(k * 16, 16)] = zeros16
        return carry

    lax.fori_loop(0, _ZC, zrow, 0)
    for k in range(_ROWS_PER_TILE // _ZC):
        pltpu.async_copy(zbuf, agg_sh.at[pl.ds(s * _ROWS_PER_TILE + k * _ZC, _ZC)],
                         ssem.at[k % _NBUF])
    for b in range(_NBUF):
        pltpu.async_copy(src_hbm.at[wid, b], sidx.at[b], sisem.at[b])
    for b in range(_LEAD):
        pltpu.async_copy(dst_hbm.at[wid, b], didx.at[b], disem.at[b])
    for b in range(_LEAD):
        pltpu.make_async_copy(src_hbm.at[wid, b], sidx.at[b], sisem.at[b]).wait()
        pltpu.async_copy(x_hbm.at[sidx.at[b]], rows.at[b], gsem.at[b])
    for k in range(_ROWS_PER_TILE // _ZC):
        pltpu.make_async_copy(zbuf, agg_sh.at[pl.ds(0, _ZC)], ssem.at[k % _NBUF]).wait()
    plsc.subcore_barrier()

    idescr = src_hbm.at[0, 0]
    def outer(g, carry):
        for b in range(_NBUF):
            j = g * _NBUF + b
            b2 = (b + _LEAD) % _NBUF
            pltpu.make_async_copy(x_hbm.at[sidx.at[b]], rows.at[b], gsem.at[b]).wait()

            @pl.when(j + _NBUF < _NCHUNK)
            def _():
                pltpu.async_copy(src_hbm.at[wid, j + _NBUF], sidx.at[b], sisem.at[b])
            pltpu.make_async_copy(idescr, didx.at[b], disem.at[b]).wait()
            pltpu.async_copy(rows.at[b], agg_sh.at[didx.at[b]], ssem.at[b], add=True)

            @pl.when(j >= _NBUF - _LEAD)
            def _():
                pltpu.make_async_copy(rows.at[b2], agg_sh.at[didx.at[0]],
                                      ssem.at[b2]).wait()

            @pl.when(j + _LEAD < _NCHUNK)
            def _():
                pltpu.async_copy(dst_hbm.at[wid, j + _LEAD], didx.at[b2], disem.at[b2])
                pltpu.make_async_copy(idescr, sidx.at[b2], sisem.at[b2]).wait()
                pltpu.async_copy(x_hbm.at[sidx.at[b2]], rows.at[b2], gsem.at[b2])
        return carry

    lax.fori_loop(0, _NCHUNK // _NBUF, outer, 0)
    for k in range(_NBUF - _LEAD):
        b = (_NCHUNK - (_NBUF - _LEAD) + k) % _NBUF
        pltpu.make_async_copy(rows.at[b], agg_sh.at[didx.at[0]], ssem.at[b]).wait()
    plsc.subcore_barrier()

    pltpu.sync_copy(
        agg_sh.at[pl.ds(s * _ROWS_PER_TILE, _ROWS_PER_TILE)],
        out_hbm.at[c, pl.ds(s * _ROWS_PER_TILE, _ROWS_PER_TILE)],
    )


_sc_gather_scatter = functools.partial(
    pl.kernel,
    out_type=jax.ShapeDtypeStruct((_NC, _APAD, _D), jnp.float32),
    mesh=plsc.VectorSubcoreMesh(core_axis_name="c", subcore_axis_name="s"),
    scratch_types=[
        pltpu.VMEM_SHARED((_APAD, _D), jnp.float32),
        pltpu.VMEM((_ZC, _D), jnp.float32),
        pltpu.VMEM((_NBUF, _CHUNK), jnp.int32),
        pltpu.VMEM((_NBUF, _CHUNK), jnp.int32),
        pltpu.VMEM((_NBUF, _CHUNK, _D), jnp.float32),
        pltpu.SemaphoreType.DMA((_NBUF,)),
        pltpu.SemaphoreType.DMA((_NBUF,)),
        pltpu.SemaphoreType.DMA((_NBUF,)),
        pltpu.SemaphoreType.DMA((_NBUF,)),
    ],
)(_sc_body)


def _mlp_body(x_ref, a_ref, w1_ref, b1_ref, w2_ref, b2_ref, o_ref, *, out_relu):
    a = x_ref[...] + a_ref[0] + a_ref[1]
    h = jnp.maximum(
        jnp.dot(a, w1_ref[...], preferred_element_type=jnp.float32) + b1_ref[...], 0.0)
    o = jnp.dot(h, w2_ref[...], preferred_element_type=jnp.float32) + b2_ref[...]
    if out_relu:
        o = jnp.maximum(o, 0.0)
    o_ref[...] = o


def _mlp(x, agg, w1, b1, w2, b2, out_relu):
    big = pl.BlockSpec((_RB, _D), lambda i: (i, 0))
    aspec = pl.BlockSpec((_NC, _RB, _D), lambda i: (0, i, 0))
    wspec = pl.BlockSpec((_D, _D), lambda i: (0, 0))
    bspec = pl.BlockSpec((1, _D), lambda i: (0, 0))
    return pl.pallas_call(
        functools.partial(_mlp_body, out_relu=out_relu),
        grid=(_NBLK,),
        in_specs=[big, aspec, wspec, bspec, wspec, bspec],
        out_specs=big,
        out_shape=jax.ShapeDtypeStruct((_N, _D), jnp.float32),
    )(x, agg, w1, b1.reshape(1, _D), w2, b2.reshape(1, _D))


def _final_body(x_ref, a_ref, w1_ref, b1_ref, w2_ref, b2_ref, bidx_ref,
                h1w_ref, h1b_ref, h2w_ref, h2b_ref, o_ref, acc, cnt):
    j = pl.program_id(0)
    a = x_ref[...] + a_ref[0] + a_ref[1]
    h = jnp.maximum(
        jnp.dot(a, w1_ref[...], preferred_element_type=jnp.float32) + b1_ref[...], 0.0)
    h3 = jnp.dot(h, w2_ref[...], preferred_element_type=jnp.float32) + b2_ref[...]

    bi = bidx_ref[0, 0, :]
    gid = lax.broadcasted_iota(jnp.int32, (_RB, _B), 1)
    oh = (bi[:, None] == gid).astype(jnp.float32)

    @pl.when(j == 0)
    def _():
        acc[...] = jnp.zeros_like(acc)
        cnt[...] = jnp.zeros_like(cnt)

    acc[...] += lax.dot_general(oh, h3, (((0,), (0,)), ((), ())),
                                preferred_element_type=jnp.float32)
    cnt[...] += jnp.sum(oh, axis=0, keepdims=True)

    @pl.when(j == pl.num_programs(0) - 1)
    def _():
        recip = 1.0 / jnp.clip(cnt[...], 1.0, None)
        eye = (lax.broadcasted_iota(jnp.int32, (_B, _B), 0) ==
               lax.broadcasted_iota(jnp.int32, (_B, _B), 1)).astype(jnp.float32)
        diag = eye * recip
        pooled = jnp.dot(diag, acc[...], preferred_element_type=jnp.float32)
        z = jnp.maximum(
            jnp.dot(pooled, h1w_ref[...], preferred_element_type=jnp.float32)
            + h1b_ref[...], 0.0)
        o_ref[...] = (jnp.dot(z, h2w_ref[...], preferred_element_type=jnp.float32)
                      + h2b_ref[...])


def _final(x, agg, w1, b1, w2, b2, bidx, h1w, h1b, h2w_pad, h2b_pad):
    big = pl.BlockSpec((_RB, _D), lambda i: (i, 0))
    aspec = pl.BlockSpec((_NC, _RB, _D), lambda i: (0, i, 0))
    wspec = pl.BlockSpec((_D, _D), lambda i: (0, 0))
    bspec = pl.BlockSpec((1, _D), lambda i: (0, 0))
    bidx_spec = pl.BlockSpec((1, 1, _RB), lambda i: (i, 0, 0))
    ospec = pl.BlockSpec((_B, _B), lambda i: (0, 0))
    return pl.pallas_call(
        _final_body,
        grid=(_NBLK,),
        in_specs=[big, aspec, wspec, bspec, wspec, bspec, bidx_spec,
                  wspec, bspec, wspec, bspec],
        out_specs=ospec,
        out_shape=jax.ShapeDtypeStruct((_B, _B), jnp.float32),
        scratch_shapes=[pltpu.VMEM((_B, _B), jnp.float32),
                        pltpu.VMEM((1, _B), jnp.float32)],
    )(x, agg, w1, b1.reshape(1, _D), w2, b2.reshape(1, _D), bidx,
      h1w, h1b.reshape(1, _D), h2w_pad, h2b_pad.reshape(1, _B))


def kernel(x, edge_index, batch_sample_indices,
           l1W1, l1b1, l1W2, l1b2,
           l2W1, l2b1, l2W2, l2b2,
           l3W1, l3b1, l3W2, l3b2,
           h1W, h1b, h2W, h2b):
    src = edge_index[0].reshape(_NW, _NCHUNK, _CHUNK)
    dst = edge_index[1].reshape(_NW, _NCHUNK, _CHUNK)
    bidx = batch_sample_indices.reshape(_NBLK, 1, _RB)
    h2w_pad = jnp.pad(h2W, ((0, 0), (0, _B - _C)))
    h2b_pad = jnp.pad(h2b, (0, _B - _C))

    agg = _sc_gather_scatter(x, src, dst)
    h = _mlp(x, agg, l1W1, l1b1, l1W2, l1b2, True)
    agg = _sc_gather_scatter(h, src, dst)
    h = _mlp(h, agg, l2W1, l2b1, l2W2, l2b2, True)
    agg = _sc_gather_scatter(h, src, dst)
    out = _final(h, agg, l3W1, l3b1, l3W2, l3b2, bidx, h1W, h1b, h2w_pad, h2b_pad)
    return out[:, :_C]
